# baseline (speedup 1.0000x reference)
import functools
import jax
import jax.numpy as jnp
from jax import lax
from jax.experimental import pallas as pl
from jax.experimental.pallas import tpu as pltpu
from jax.experimental.pallas import tpu_sc as plsc

N = 10000
E = 320000
F_IN = 128
HID = 8
HEADS = 8
NCLS = 7

NC = 2
NS = 16
NW = NC * NS
C = 125
EPT = E // NW
NCH = EPT // C
ZCH = 125
STRIPE = 625
NACC = N
TW = 72
NBUF_B = 2
NBUF_D = 2

_mesh = plsc.VectorSubcoreMesh(
    core_axis_name="c", subcore_axis_name="s", num_cores=NC, num_subcores=NS)

_sc_params = pltpu.CompilerParams(
    use_tc_tiling_on_sc=False, needs_layout_passes=False)


def _mm_tables_body(x_ref, w_ref, tbl_ref, adt_ref):
  y = jnp.dot(x_ref[...], w_ref[...], preferred_element_type=jnp.float32)
  tbl_ref[...] = y[:, :TW]
  adt_ref[...] = y[:, TW:]


def _build_tables1(x, wcat):
  blk = 2000
  return pl.pallas_call(
      _mm_tables_body,
      grid=(N // blk,),
      in_specs=[
          pl.BlockSpec((blk, F_IN), lambda i: (i, 0)),
          pl.BlockSpec((F_IN, TW + 16), lambda i: (0, 0)),
      ],
      out_specs=[
          pl.BlockSpec((blk, TW), lambda i: (i, 0)),
          pl.BlockSpec((blk, 16), lambda i: (i, 0)),
      ],
      out_shape=[
          jax.ShapeDtypeStruct((N, TW), jnp.float32),
          jax.ShapeDtypeStruct((N, 16), jnp.float32),
      ],
  )(x, wcat)


def _edge_pass1(tbl1, adt1, src2d, dst2d):
  @functools.partial(
      pl.kernel,
      out_type=jax.ShapeDtypeStruct((NC, NACC, TW), jnp.float32),
      mesh=_mesh,
      compiler_params=_sc_params,
      scratch_types=[
          pltpu.VMEM((NCH, C), jnp.int32),
          pltpu.VMEM((NCH, C), jnp.int32),
          *([pltpu.VMEM((C, TW), jnp.float32)] * NBUF_B),
          *([pltpu.VMEM((C, 16), jnp.float32)] * NBUF_B),
          *([pltpu.VMEM((C, TW), jnp.float32)] * NBUF_B),
          pltpu.VMEM((ZCH, TW), jnp.float32),
          pltpu.VMEM_SHARED((NACC, TW), jnp.float32),
          *([pltpu.SemaphoreType.DMA] * (3 * NBUF_B)),
      ],
  )
  def k(tbl_hbm, adt_hbm, src_hbm, dst_hbm, out_hbm,
        *refs):
    src_v, dst_v = refs[0], refs[1]
    rows = refs[2:2 + NBUF_B]
    ads = refs[2 + NBUF_B:2 + 2 * NBUF_B]
    msgs = refs[2 + 2 * NBUF_B:2 + 3 * NBUF_B]
    exp_v = refs[2 + 3 * NBUF_B]
    acc_sh = refs[3 + 3 * NBUF_B]
    sgr = refs[4 + 3 * NBUF_B:4 + 4 * NBUF_B]
    sga = refs[4 + 4 * NBUF_B:4 + 5 * NBUF_B]
    ss = refs[4 + 5 * NBUF_B:4 + 6 * NBUF_B]
    cid = lax.axis_index("c")
    sid = lax.axis_index("s")
    wid = sid * NC + cid
    lanes = lax.iota(jnp.int32, 16)
    idx_j = [(lanes >> 3) + 2 * j for j in range(4)]

    def zbody(i, carry):
      for off in (0, 16, 32, 48, 56):
        exp_v[i, pl.ds(off, 16)] = jnp.zeros((16,), jnp.float32)
      return carry
    lax.fori_loop(0, ZCH, zbody, 0)
    for kk in range(STRIPE // ZCH):
      pltpu.sync_copy(exp_v, acc_sh.at[pl.ds(sid * STRIPE + kk * ZCH, ZCH)])
    plsc.subcore_barrier()

    pltpu.sync_copy(src_hbm.at[wid], src_v)
    pltpu.sync_copy(dst_hbm.at[wid], dst_v)

    def issue_gather(ch, b):
      pltpu.async_copy(tbl_hbm.at[src_v.at[ch]], rows[b], sgr[b])
      pltpu.async_copy(adt_hbm.at[dst_v.at[ch]], ads[b], sga[b])

    def wait_gather(b):
      pltpu.make_async_copy(tbl_hbm.at[src_v.at[0]], rows[b], sgr[b]).wait()
      pltpu.make_async_copy(adt_hbm.at[dst_v.at[0]], ads[b], sga[b]).wait()

    def wait_scatter(b):
      pltpu.make_async_copy(msgs[b], acc_sh.at[dst_v.at[0]], ss[b]).wait()

    def compute_chunk(b):
      @plsc.parallel_loop(0, C, 1, unroll=5)
      def ebody(i):
        e = rows[b][i, pl.ds(0, 16)] + ads[b][i, :]
        ex = jnp.exp(jnp.maximum(e, 0.2 * e))
        msgs[b][i, pl.ds(0, 16)] = ex
        row16 = lanes * 0 + i
        for j in range(4):
          exj = plsc.load_gather(msgs[b], [row16, idx_j[j]])
          msgs[b][i, pl.ds(8 + 16 * j, 16)] = (
              rows[b][i, pl.ds(8 + 16 * j, 16)] * exj)

    for b in range(NBUF_B - 1):
      issue_gather(b, b)
    ngrp = NCH // NBUF_B

    def grp_body(g, carry):
      base = g * NBUF_B
      for b in range(NBUF_B):
        wait_gather(b)

        @pl.when(g > 0)
        def _():
          wait_scatter(b)

        compute_chunk(b)
        pltpu.async_copy(msgs[b], acc_sh.at[dst_v.at[base + b]], ss[b],
                         add=True)

        @pl.when(base + b + NBUF_B - 1 < NCH)
        def _():
          issue_gather(base + b + NBUF_B - 1, (b + NBUF_B - 1) % NBUF_B)
      return carry
    lax.fori_loop(0, ngrp, grp_body, 0)
    for b in range(NBUF_B):
      wait_scatter(b)

    plsc.subcore_barrier()

    for kk in range(STRIPE // ZCH):
      base = sid * STRIPE + kk * ZCH
      pltpu.sync_copy(acc_sh.at[pl.ds(base, ZCH)], exp_v)
      pltpu.sync_copy(exp_v, out_hbm.at[cid].at[pl.ds(base, ZCH)])

  return k(tbl1, adt1, src2d, dst2d)


def _layer1_finish_body(acc_ref, tbl_ref, adt_ref, w2e_ref, b1_ref, r_ref,
                        tbl2_ref, adt2_ref):
  a = acc_ref[0] + acc_ref[1]
  as8 = tbl_ref[:, :8]
  h = tbl_ref[:, 8:]
  e = as8 + adt_ref[:, :8]
  ex8 = jnp.exp(jnp.maximum(e, 0.2 * e))
  den8 = a[:, :8] + ex8
  rmat = r_ref[...]
  exb = jnp.dot(ex8, rmat, preferred_element_type=jnp.float32)
  denb = jnp.dot(den8, rmat, preferred_element_type=jnp.float32)
  num = a[:, 8:] + h * exb
  out1 = num / (denb + 1e-16) + b1_ref[...]
  h2 = jnp.where(out1 > 0, out1, jnp.exp(out1) - 1.0)
  zc = jnp.dot(h2, w2e_ref[...], preferred_element_type=jnp.float32)
  z = zc[:, :NCLS]
  as2 = zc[:, NCLS:NCLS + 1]
  ad2 = zc[:, NCLS + 1:NCLS + 2]
  one = jnp.ones_like(as2)
  zero7 = jnp.zeros_like(z)
  tbl2_ref[...] = jnp.concatenate([one, z, as2, zero7], axis=1)
  adt2_ref[...] = ad2 * jnp.ones((1, 16), jnp.float32)


def _layer1_finish(acc1, tbl1, adt1, w2e, b1, rmat):
  blk = 2000
  return pl.pallas_call(
      _layer1_finish_body,
      grid=(N // blk,),
      in_specs=[
          pl.BlockSpec((NC, blk, TW), lambda i: (0, i, 0)),
          pl.BlockSpec((blk, TW), lambda i: (i, 0)),
          pl.BlockSpec((blk, 16), lambda i: (i, 0)),
          pl.BlockSpec((64, NCLS + 2), lambda i: (0, 0)),
          pl.BlockSpec((1, 64), lambda i: (0, 0)),
          pl.BlockSpec((8, 64), lambda i: (0, 0)),
      ],
      out_specs=[
          pl.BlockSpec((blk, 16), lambda i: (i, 0)),
          pl.BlockSpec((blk, 16), lambda i: (i, 0)),
      ],
      out_shape=[
          jax.ShapeDtypeStruct((N, 16), jnp.float32),
          jax.ShapeDtypeStruct((N, 16), jnp.float32),
      ],
  )(acc1, tbl1, adt1, w2e, b1, rmat)


def _edge_pass2(tbl2, adt2, src2d, dst2d):
  @functools.partial(
      pl.kernel,
      out_type=jax.ShapeDtypeStruct((NC, NACC, 16), jnp.float32),
      mesh=_mesh,
      compiler_params=_sc_params,
      scratch_types=[
          pltpu.VMEM((NCH, C), jnp.int32),
          pltpu.VMEM((NCH, C), jnp.int32),
          *([pltpu.VMEM((C, 16), jnp.float32)] * (3 * NBUF_D)),
          pltpu.VMEM((ZCH, 16), jnp.float32),
          pltpu.VMEM_SHARED((NACC, 16), jnp.float32),
          *([pltpu.SemaphoreType.DMA] * (3 * NBUF_D)),
      ],
  )
  def k(tbl_hbm, adt_hbm, src_hbm, dst_hbm, out_hbm,
        *refs):
    src_v, dst_v = refs[0], refs[1]
    rows = refs[2:2 + NBUF_D]
    ads = refs[2 + NBUF_D:2 + 2 * NBUF_D]
    msgs = refs[2 + 2 * NBUF_D:2 + 3 * NBUF_D]
    exp_v = refs[2 + 3 * NBUF_D]
    acc_sh = refs[3 + 3 * NBUF_D]
    sgr = refs[4 + 3 * NBUF_D:4 + 4 * NBUF_D]
    sga = refs[4 + 4 * NBUF_D:4 + 5 * NBUF_D]
    ss = refs[4 + 5 * NBUF_D:4 + 6 * NBUF_D]
    cid = lax.axis_index("c")
    sid = lax.axis_index("s")
    wid = sid * NC + cid
    lanes = lax.iota(jnp.int32, 16)
    idx8 = lanes * 0 + 8

    def zbody(i, carry):
      exp_v[i, :] = jnp.zeros((16,), jnp.float32)
      return carry
    lax.fori_loop(0, ZCH, zbody, 0)
    for kk in range(STRIPE // ZCH):
      pltpu.sync_copy(exp_v, acc_sh.at[pl.ds(sid * STRIPE + kk * ZCH, ZCH)])
    plsc.subcore_barrier()

    pltpu.sync_copy(src_hbm.at[wid], src_v)
    pltpu.sync_copy(dst_hbm.at[wid], dst_v)

    def issue_gather(ch, b):
      pltpu.async_copy(tbl_hbm.at[src_v.at[ch]], rows[b], sgr[b])
      pltpu.async_copy(adt_hbm.at[dst_v.at[ch]], ads[b], sga[b])

    def wait_gather(b):
      pltpu.make_async_copy(tbl_hbm.at[src_v.at[0]], rows[b], sgr[b]).wait()
      pltpu.make_async_copy(adt_hbm.at[dst_v.at[0]], ads[b], sga[b]).wait()

    def wait_scatter(b):
      pltpu.make_async_copy(msgs[b], acc_sh.at[dst_v.at[0]], ss[b]).wait()

    def compute_chunk(b):
      @plsc.parallel_loop(0, C, 1, unroll=5)
      def ebody(i):
        u = rows[b][i, :]
        as2 = plsc.load_gather(rows[b], [lanes * 0 + i, idx8])
        e = as2 + ads[b][i, :]
        ex = jnp.exp(jnp.maximum(e, 0.2 * e))
        msgs[b][i, :] = ex * u

    for b in range(NBUF_D - 1):
      issue_gather(b, b)
    ngrp = NCH // NBUF_D

    def grp_body(g, carry):
      base = g * NBUF_D
      for b in range(NBUF_D):
        wait_gather(b)

        @pl.when(g > 0)
        def _():
          wait_scatter(b)

        compute_chunk(b)
        pltpu.async_copy(msgs[b], acc_sh.at[dst_v.at[base + b]], ss[b],
                         add=True)

        @pl.when(base + b + NBUF_D - 1 < NCH)
        def _():
          issue_gather(base + b + NBUF_D - 1, (b + NBUF_D - 1) % NBUF_D)
      return carry
    lax.fori_loop(0, ngrp, grp_body, 0)
    for b in range(NBUF_D):
      wait_scatter(b)

    plsc.subcore_barrier()

    for kk in range(STRIPE // ZCH):
      base = sid * STRIPE + kk * ZCH
      pltpu.sync_copy(acc_sh.at[pl.ds(base, ZCH)], exp_v)
      pltpu.sync_copy(exp_v, out_hbm.at[cid].at[pl.ds(base, ZCH)])

  return k(tbl2, adt2, src2d, dst2d)


def _final_body(acc_ref, tbl2_ref, adt2_ref, b2_ref, out_ref):
  a = acc_ref[0] + acc_ref[1]
  e = tbl2_ref[:, 8:9] + adt2_ref[:, 0:1]
  ex = jnp.exp(jnp.maximum(e, 0.2 * e))
  a = a + ex * tbl2_ref[:, :16]
  den = a[:, 0:1]
  num = a[:, 1:8]
  out2 = num / (den + 1e-16) + b2_ref[...]
  m = jnp.max(out2, axis=1, keepdims=True)
  lse = jnp.log(jnp.sum(jnp.exp(out2 - m), axis=1, keepdims=True)) + m
  out_ref[...] = out2 - lse


def _final(acc2, tbl2, adt2, b2):
  blk = 2000
  return pl.pallas_call(
      _final_body,
      grid=(N // blk,),
      in_specs=[
          pl.BlockSpec((NC, blk, 16), lambda i: (0, i, 0)),
          pl.BlockSpec((blk, 16), lambda i: (i, 0)),
          pl.BlockSpec((blk, 16), lambda i: (i, 0)),
          pl.BlockSpec((1, NCLS), lambda i: (0, 0)),
      ],
      out_specs=pl.BlockSpec((blk, NCLS), lambda i: (i, 0)),
      out_shape=jax.ShapeDtypeStruct((N, NCLS), jnp.float32),
  )(acc2, tbl2, adt2, b2)


def kernel(x, edge_index, W1, a1_src, a1_dst, b1, W2, a2_src, a2_dst, b2):
  eyeh = jnp.eye(HEADS, dtype=jnp.float32)
  a_src = (a1_src[:, :, None] * eyeh[:, None, :]).reshape(64, 8)
  a_dst = (a1_dst[:, :, None] * eyeh[:, None, :]).reshape(64, 8)
  zpad = jnp.zeros((F_IN, 8), jnp.float32)
  wcat = jnp.concatenate(
      [W1 @ a_src, W1, W1 @ a_dst, zpad], axis=1)
  rmat = jnp.kron(eyeh, jnp.ones((1, HID), jnp.float32))
  w2e = jnp.concatenate([W2, W2 @ a2_src.T, W2 @ a2_dst.T], axis=1)

  src2d = edge_index[0].astype(jnp.int32).reshape(NW, NCH, C)
  dst2d = edge_index[1].astype(jnp.int32).reshape(NW, NCH, C)
  src2d, dst2d = lax.optimization_barrier((src2d, dst2d))

  tbl1, adt1 = _build_tables1(x, wcat)
  acc1 = _edge_pass1(tbl1, adt1, src2d, dst2d)
  tbl2, adt2 = _layer1_finish(acc1, tbl1, adt1, w2e, b1.reshape(1, 64), rmat)
  acc2 = _edge_pass2(tbl2, adt2, src2d, dst2d)
  return _final(acc2, tbl2, adt2, b2.reshape(1, NCLS))


# speedup vs baseline: 1.4011x; 1.4011x over previous
import functools
import jax
import jax.numpy as jnp
from jax import lax
from jax.experimental import pallas as pl
from jax.experimental.pallas import tpu as pltpu
from jax.experimental.pallas import tpu_sc as plsc

N = 10000
E = 320000
F_IN = 128
HID = 8
HEADS = 8
NCLS = 7

NC = 2
NS = 16
NW = NC * NS
C = 125
EPT = E // NW
NCH = EPT // C
ZCH = 125
STRIPE = 625
NACC = N
TW = 72
NBUF_B = 2
NBUF_D = 2

_mesh = plsc.VectorSubcoreMesh(
    core_axis_name="c", subcore_axis_name="s", num_cores=NC, num_subcores=NS)

_sc_params = pltpu.CompilerParams(
    use_tc_tiling_on_sc=False, needs_layout_passes=False)


def _mm_tables_body(x_ref, w_ref, tbl_ref, adt_ref):
  y = jnp.dot(x_ref[...], w_ref[...], preferred_element_type=jnp.float32)
  tbl_ref[...] = y[:, :TW]
  adt_ref[...] = y[:, TW:]


def _build_tables1(x, wcat):
  blk = 2000
  return pl.pallas_call(
      _mm_tables_body,
      grid=(N // blk,),
      in_specs=[
          pl.BlockSpec((blk, F_IN), lambda i: (i, 0)),
          pl.BlockSpec((F_IN, TW + 16), lambda i: (0, 0)),
      ],
      out_specs=[
          pl.BlockSpec((blk, TW), lambda i: (i, 0)),
          pl.BlockSpec((blk, 16), lambda i: (i, 0)),
      ],
      out_shape=[
          jax.ShapeDtypeStruct((N, TW), jnp.float32),
          jax.ShapeDtypeStruct((N, 16), jnp.float32),
      ],
  )(x, wcat)


def _edge_pass1(tbl1, adt1, src2d, dst2d):
  @functools.partial(
      pl.kernel,
      out_type=jax.ShapeDtypeStruct((NC, NACC, TW), jnp.float32),
      mesh=_mesh,
      compiler_params=_sc_params,
      scratch_types=[
          pltpu.VMEM((NCH, C), jnp.int32),
          pltpu.VMEM((NCH, C), jnp.int32),
          *([pltpu.VMEM((C, TW), jnp.float32)] * NBUF_B),
          *([pltpu.VMEM((C, 16), jnp.float32)] * NBUF_B),
          *([pltpu.VMEM((C, TW), jnp.float32)] * NBUF_B),
          pltpu.VMEM((ZCH, TW), jnp.float32),
          pltpu.VMEM_SHARED((NACC, TW), jnp.float32),
          *([pltpu.SemaphoreType.DMA] * (3 * NBUF_B)),
      ],
  )
  def k(tbl_hbm, adt_hbm, src_hbm, dst_hbm, out_hbm,
        *refs):
    src_v, dst_v = refs[0], refs[1]
    rows = refs[2:2 + NBUF_B]
    ads = refs[2 + NBUF_B:2 + 2 * NBUF_B]
    msgs = refs[2 + 2 * NBUF_B:2 + 3 * NBUF_B]
    exp_v = refs[2 + 3 * NBUF_B]
    acc_sh = refs[3 + 3 * NBUF_B]
    sgr = refs[4 + 3 * NBUF_B:4 + 4 * NBUF_B]
    sga = refs[4 + 4 * NBUF_B:4 + 5 * NBUF_B]
    ss = refs[4 + 5 * NBUF_B:4 + 6 * NBUF_B]
    cid = lax.axis_index("c")
    sid = lax.axis_index("s")
    wid = sid * NC + cid
    lanes = lax.iota(jnp.int32, 16)
    idx_j = [(lanes >> 3) + 2 * j for j in range(4)]

    def zbody(i, carry):
      for off in (0, 16, 32, 48, 56):
        exp_v[i, pl.ds(off, 16)] = jnp.zeros((16,), jnp.float32)
      return carry
    lax.fori_loop(0, ZCH, zbody, 0)
    for kk in range(STRIPE // ZCH):
      pltpu.sync_copy(exp_v, acc_sh.at[pl.ds(sid * STRIPE + kk * ZCH, ZCH)])
    plsc.subcore_barrier()

    pltpu.sync_copy(src_hbm.at[wid], src_v)
    pltpu.sync_copy(dst_hbm.at[wid], dst_v)

    def issue_gather(ch, b):
      pltpu.async_copy(tbl_hbm.at[src_v.at[ch]], rows[b], sgr[b])
      pltpu.async_copy(adt_hbm.at[dst_v.at[ch]], ads[b], sga[b])

    def wait_gather(b):
      pltpu.make_async_copy(tbl_hbm.at[src_v.at[0]], rows[b], sgr[b]).wait()
      pltpu.make_async_copy(adt_hbm.at[dst_v.at[0]], ads[b], sga[b]).wait()

    def wait_scatter(b):
      pltpu.make_async_copy(msgs[b], acc_sh.at[dst_v.at[0]], ss[b]).wait()

    def compute_chunk(b):
      @plsc.parallel_loop(0, C, 1, unroll=5)
      def ebody(i):
        e = rows[b][i, pl.ds(0, 16)] + ads[b][i, :]
        ex = jnp.exp(jnp.maximum(e, 0.2 * e))
        msgs[b][i, pl.ds(0, 16)] = ex
        row16 = lanes * 0 + i
        for j in range(4):
          exj = plsc.load_gather(msgs[b], [row16, idx_j[j]])
          msgs[b][i, pl.ds(8 + 16 * j, 16)] = (
              rows[b][i, pl.ds(8 + 16 * j, 16)] * exj)

    for b in range(NBUF_B):
      issue_gather(b, b)
    ngrp = NCH // NBUF_B

    def grp_body(g, carry):
      base = g * NBUF_B
      for b in range(NBUF_B):
        wait_gather(b)

        @pl.when(g > 0)
        def _():
          wait_scatter(b)

        compute_chunk(b)
        pltpu.async_copy(msgs[b], acc_sh.at[dst_v.at[base + b]], ss[b],
                         add=True)

        @pl.when(base + b + NBUF_B < NCH)
        def _():
          issue_gather(base + b + NBUF_B, b)
      return carry
    lax.fori_loop(0, ngrp, grp_body, 0)
    for b in range(NBUF_B):
      wait_scatter(b)

    plsc.subcore_barrier()

    for kk in range(STRIPE // ZCH):
      base = sid * STRIPE + kk * ZCH
      pltpu.sync_copy(acc_sh.at[pl.ds(base, ZCH)], exp_v)
      pltpu.sync_copy(exp_v, out_hbm.at[cid].at[pl.ds(base, ZCH)])

  return k(tbl1, adt1, src2d, dst2d)


def _layer1_finish_body(acc_ref, tbl_ref, adt_ref, w2e_ref, b1_ref, r_ref,
                        tbl2_ref, adt2_ref):
  a = acc_ref[0] + acc_ref[1]
  as8 = tbl_ref[:, :8]
  h = tbl_ref[:, 8:]
  e = as8 + adt_ref[:, :8]
  ex8 = jnp.exp(jnp.maximum(e, 0.2 * e))
  den8 = a[:, :8] + ex8
  rmat = r_ref[...]
  exb = jnp.dot(ex8, rmat, preferred_element_type=jnp.float32)
  denb = jnp.dot(den8, rmat, preferred_element_type=jnp.float32)
  num = a[:, 8:] + h * exb
  out1 = num / (denb + 1e-16) + b1_ref[...]
  h2 = jnp.where(out1 > 0, out1, jnp.exp(out1) - 1.0)
  zc = jnp.dot(h2, w2e_ref[...], preferred_element_type=jnp.float32)
  z = zc[:, :NCLS]
  as2 = zc[:, NCLS:NCLS + 1]
  ad2 = zc[:, NCLS + 1:NCLS + 2]
  one = jnp.ones_like(as2)
  zero7 = jnp.zeros_like(z)
  tbl2_ref[...] = jnp.concatenate([one, z, as2, zero7], axis=1)
  adt2_ref[...] = ad2 * jnp.ones((1, 16), jnp.float32)


def _layer1_finish(acc1, tbl1, adt1, w2e, b1, rmat):
  blk = 2000
  return pl.pallas_call(
      _layer1_finish_body,
      grid=(N // blk,),
      in_specs=[
          pl.BlockSpec((NC, blk, TW), lambda i: (0, i, 0)),
          pl.BlockSpec((blk, TW), lambda i: (i, 0)),
          pl.BlockSpec((blk, 16), lambda i: (i, 0)),
          pl.BlockSpec((64, NCLS + 2), lambda i: (0, 0)),
          pl.BlockSpec((1, 64), lambda i: (0, 0)),
          pl.BlockSpec((8, 64), lambda i: (0, 0)),
      ],
      out_specs=[
          pl.BlockSpec((blk, 16), lambda i: (i, 0)),
          pl.BlockSpec((blk, 16), lambda i: (i, 0)),
      ],
      out_shape=[
          jax.ShapeDtypeStruct((N, 16), jnp.float32),
          jax.ShapeDtypeStruct((N, 16), jnp.float32),
      ],
  )(acc1, tbl1, adt1, w2e, b1, rmat)


def _edge_pass2(tbl2, adt2, src2d, dst2d):
  @functools.partial(
      pl.kernel,
      out_type=jax.ShapeDtypeStruct((NC, NACC, 16), jnp.float32),
      mesh=_mesh,
      compiler_params=_sc_params,
      scratch_types=[
          pltpu.VMEM((NCH, C), jnp.int32),
          pltpu.VMEM((NCH, C), jnp.int32),
          *([pltpu.VMEM((C, 16), jnp.float32)] * (3 * NBUF_D)),
          pltpu.VMEM((ZCH, 16), jnp.float32),
          pltpu.VMEM_SHARED((NACC, 16), jnp.float32),
          *([pltpu.SemaphoreType.DMA] * (3 * NBUF_D)),
      ],
  )
  def k(tbl_hbm, adt_hbm, src_hbm, dst_hbm, out_hbm,
        *refs):
    src_v, dst_v = refs[0], refs[1]
    rows = refs[2:2 + NBUF_D]
    ads = refs[2 + NBUF_D:2 + 2 * NBUF_D]
    msgs = refs[2 + 2 * NBUF_D:2 + 3 * NBUF_D]
    exp_v = refs[2 + 3 * NBUF_D]
    acc_sh = refs[3 + 3 * NBUF_D]
    sgr = refs[4 + 3 * NBUF_D:4 + 4 * NBUF_D]
    sga = refs[4 + 4 * NBUF_D:4 + 5 * NBUF_D]
    ss = refs[4 + 5 * NBUF_D:4 + 6 * NBUF_D]
    cid = lax.axis_index("c")
    sid = lax.axis_index("s")
    wid = sid * NC + cid
    lanes = lax.iota(jnp.int32, 16)
    idx8 = lanes * 0 + 8

    def zbody(i, carry):
      exp_v[i, :] = jnp.zeros((16,), jnp.float32)
      return carry
    lax.fori_loop(0, ZCH, zbody, 0)
    for kk in range(STRIPE // ZCH):
      pltpu.sync_copy(exp_v, acc_sh.at[pl.ds(sid * STRIPE + kk * ZCH, ZCH)])
    plsc.subcore_barrier()

    pltpu.sync_copy(src_hbm.at[wid], src_v)
    pltpu.sync_copy(dst_hbm.at[wid], dst_v)

    def issue_gather(ch, b):
      pltpu.async_copy(tbl_hbm.at[src_v.at[ch]], rows[b], sgr[b])
      pltpu.async_copy(adt_hbm.at[dst_v.at[ch]], ads[b], sga[b])

    def wait_gather(b):
      pltpu.make_async_copy(tbl_hbm.at[src_v.at[0]], rows[b], sgr[b]).wait()
      pltpu.make_async_copy(adt_hbm.at[dst_v.at[0]], ads[b], sga[b]).wait()

    def wait_scatter(b):
      pltpu.make_async_copy(msgs[b], acc_sh.at[dst_v.at[0]], ss[b]).wait()

    def compute_chunk(b):
      @plsc.parallel_loop(0, C, 1, unroll=5)
      def ebody(i):
        u = rows[b][i, :]
        as2 = plsc.load_gather(rows[b], [lanes * 0 + i, idx8])
        e = as2 + ads[b][i, :]
        ex = jnp.exp(jnp.maximum(e, 0.2 * e))
        msgs[b][i, :] = ex * u

    for b in range(NBUF_D):
      issue_gather(b, b)
    ngrp = NCH // NBUF_D

    def grp_body(g, carry):
      base = g * NBUF_D
      for b in range(NBUF_D):
        wait_gather(b)

        @pl.when(g > 0)
        def _():
          wait_scatter(b)

        compute_chunk(b)
        pltpu.async_copy(msgs[b], acc_sh.at[dst_v.at[base + b]], ss[b],
                         add=True)

        @pl.when(base + b + NBUF_D < NCH)
        def _():
          issue_gather(base + b + NBUF_D, b)
      return carry
    lax.fori_loop(0, ngrp, grp_body, 0)
    for b in range(NBUF_D):
      wait_scatter(b)

    plsc.subcore_barrier()

    for kk in range(STRIPE // ZCH):
      base = sid * STRIPE + kk * ZCH
      pltpu.sync_copy(acc_sh.at[pl.ds(base, ZCH)], exp_v)
      pltpu.sync_copy(exp_v, out_hbm.at[cid].at[pl.ds(base, ZCH)])

  return k(tbl2, adt2, src2d, dst2d)


def _final_body(acc_ref, tbl2_ref, adt2_ref, b2_ref, out_ref):
  a = acc_ref[0] + acc_ref[1]
  e = tbl2_ref[:, 8:9] + adt2_ref[:, 0:1]
  ex = jnp.exp(jnp.maximum(e, 0.2 * e))
  a = a + ex * tbl2_ref[:, :16]
  den = a[:, 0:1]
  num = a[:, 1:8]
  out2 = num / (den + 1e-16) + b2_ref[...]
  m = jnp.max(out2, axis=1, keepdims=True)
  lse = jnp.log(jnp.sum(jnp.exp(out2 - m), axis=1, keepdims=True)) + m
  out_ref[...] = out2 - lse


def _final(acc2, tbl2, adt2, b2):
  blk = 2000
  return pl.pallas_call(
      _final_body,
      grid=(N // blk,),
      in_specs=[
          pl.BlockSpec((NC, blk, 16), lambda i: (0, i, 0)),
          pl.BlockSpec((blk, 16), lambda i: (i, 0)),
          pl.BlockSpec((blk, 16), lambda i: (i, 0)),
          pl.BlockSpec((1, NCLS), lambda i: (0, 0)),
      ],
      out_specs=pl.BlockSpec((blk, NCLS), lambda i: (i, 0)),
      out_shape=jax.ShapeDtypeStruct((N, NCLS), jnp.float32),
  )(acc2, tbl2, adt2, b2)


def kernel(x, edge_index, W1, a1_src, a1_dst, b1, W2, a2_src, a2_dst, b2):
  eyeh = jnp.eye(HEADS, dtype=jnp.float32)
  a_src = (a1_src[:, :, None] * eyeh[:, None, :]).reshape(64, 8)
  a_dst = (a1_dst[:, :, None] * eyeh[:, None, :]).reshape(64, 8)
  zpad = jnp.zeros((F_IN, 8), jnp.float32)
  wcat = jnp.concatenate(
      [W1 @ a_src, W1, W1 @ a_dst, zpad], axis=1)
  rmat = jnp.kron(eyeh, jnp.ones((1, HID), jnp.float32))
  w2e = jnp.concatenate([W2, W2 @ a2_src.T, W2 @ a2_dst.T], axis=1)

  src2d = edge_index[0].astype(jnp.int32).reshape(NW, NCH, C)
  dst2d = edge_index[1].astype(jnp.int32).reshape(NW, NCH, C)
  src2d, dst2d = lax.optimization_barrier((src2d, dst2d))

  tbl1, adt1 = _build_tables1(x, wcat)
  acc1 = _edge_pass1(tbl1, adt1, src2d, dst2d)
  tbl2, adt2 = _layer1_finish(acc1, tbl1, adt1, w2e, b1.reshape(1, 64), rmat)
  acc2 = _edge_pass2(tbl2, adt2, src2d, dst2d)
  return _final(acc2, tbl2, adt2, b2.reshape(1, NCLS))


# speedup vs baseline: 1.5096x; 1.0775x over previous
import functools
import jax
import jax.numpy as jnp
from jax import lax
from jax.experimental import pallas as pl
from jax.experimental.pallas import tpu as pltpu
from jax.experimental.pallas import tpu_sc as plsc

N = 10000
E = 320000
F_IN = 128
HID = 8
HEADS = 8
NCLS = 7

NC = 2
NS = 16
NW = NC * NS
C = 125
EPT = E // NW
NCH = EPT // C
ZCH = 125
STRIPE = 625
NACC = N
TW = 72
NBUF_B = 2
NBUF_D = 4

_mesh = plsc.VectorSubcoreMesh(
    core_axis_name="c", subcore_axis_name="s", num_cores=NC, num_subcores=NS)

_sc_params = pltpu.CompilerParams(
    use_tc_tiling_on_sc=False, needs_layout_passes=False)


def _mm_tables_body(x_ref, w_ref, tbl_ref, adt_ref):
  y = jnp.dot(x_ref[...], w_ref[...], preferred_element_type=jnp.float32)
  tbl_ref[...] = y[:, :TW]
  adt_ref[...] = y[:, TW:]


def _build_tables1(x, wcat):
  blk = 2000
  return pl.pallas_call(
      _mm_tables_body,
      grid=(N // blk,),
      in_specs=[
          pl.BlockSpec((blk, F_IN), lambda i: (i, 0)),
          pl.BlockSpec((F_IN, TW + 16), lambda i: (0, 0)),
      ],
      out_specs=[
          pl.BlockSpec((blk, TW), lambda i: (i, 0)),
          pl.BlockSpec((blk, 16), lambda i: (i, 0)),
      ],
      out_shape=[
          jax.ShapeDtypeStruct((N, TW), jnp.float32),
          jax.ShapeDtypeStruct((N, 16), jnp.float32),
      ],
  )(x, wcat)


def _edge_pass1(tbl1, adt1, src2d, dst2d):
  @functools.partial(
      pl.kernel,
      out_type=jax.ShapeDtypeStruct((NC, NACC, TW), jnp.float32),
      mesh=_mesh,
      compiler_params=_sc_params,
      scratch_types=[
          pltpu.VMEM((NCH, C), jnp.int32),
          pltpu.VMEM((NCH, C), jnp.int32),
          *([pltpu.VMEM((C, TW), jnp.float32)] * NBUF_B),
          *([pltpu.VMEM((C, 16), jnp.float32)] * NBUF_B),
          *([pltpu.VMEM((C, TW), jnp.float32)] * NBUF_B),
          pltpu.VMEM((ZCH, TW), jnp.float32),
          pltpu.VMEM_SHARED((NACC, TW), jnp.float32),
          *([pltpu.SemaphoreType.DMA] * (3 * NBUF_B)),
      ],
  )
  def k(tbl_hbm, adt_hbm, src_hbm, dst_hbm, out_hbm,
        *refs):
    src_v, dst_v = refs[0], refs[1]
    rows = refs[2:2 + NBUF_B]
    ads = refs[2 + NBUF_B:2 + 2 * NBUF_B]
    msgs = refs[2 + 2 * NBUF_B:2 + 3 * NBUF_B]
    exp_v = refs[2 + 3 * NBUF_B]
    acc_sh = refs[3 + 3 * NBUF_B]
    sgr = refs[4 + 3 * NBUF_B:4 + 4 * NBUF_B]
    sga = refs[4 + 4 * NBUF_B:4 + 5 * NBUF_B]
    ss = refs[4 + 5 * NBUF_B:4 + 6 * NBUF_B]
    cid = lax.axis_index("c")
    sid = lax.axis_index("s")
    wid = sid * NC + cid
    lanes = lax.iota(jnp.int32, 16)
    idx_j = [(lanes >> 3) + 2 * j for j in range(4)]

    def zbody(i, carry):
      for off in (0, 16, 32, 48, 56):
        exp_v[i, pl.ds(off, 16)] = jnp.zeros((16,), jnp.float32)
      return carry
    lax.fori_loop(0, ZCH, zbody, 0)
    for kk in range(STRIPE // ZCH):
      pltpu.sync_copy(exp_v, acc_sh.at[pl.ds(sid * STRIPE + kk * ZCH, ZCH)])
    plsc.subcore_barrier()

    pltpu.sync_copy(src_hbm.at[wid], src_v)
    pltpu.sync_copy(dst_hbm.at[wid], dst_v)

    def issue_gather(ch, b):
      pltpu.async_copy(tbl_hbm.at[src_v.at[ch]], rows[b], sgr[b])
      pltpu.async_copy(adt_hbm.at[dst_v.at[ch]], ads[b], sga[b])

    def wait_gather(b):
      pltpu.make_async_copy(tbl_hbm.at[src_v.at[0]], rows[b], sgr[b]).wait()
      pltpu.make_async_copy(adt_hbm.at[dst_v.at[0]], ads[b], sga[b]).wait()

    def wait_scatter(b):
      pltpu.make_async_copy(msgs[b], acc_sh.at[dst_v.at[0]], ss[b]).wait()

    def compute_chunk(b):
      @plsc.parallel_loop(0, C, 1, unroll=5)
      def ebody(i):
        e = rows[b][i, pl.ds(0, 16)] + ads[b][i, :]
        ex = jnp.exp(jnp.maximum(e, 0.2 * e))
        msgs[b][i, pl.ds(0, 16)] = ex
        row16 = lanes * 0 + i
        for j in range(4):
          exj = plsc.load_gather(msgs[b], [row16, idx_j[j]])
          msgs[b][i, pl.ds(8 + 16 * j, 16)] = (
              rows[b][i, pl.ds(8 + 16 * j, 16)] * exj)

    for b in range(NBUF_B):
      issue_gather(b, b)
    ngrp = NCH // NBUF_B

    def grp_body(g, carry):
      base = g * NBUF_B
      for b in range(NBUF_B):
        wait_gather(b)

        @pl.when(g > 0)
        def _():
          wait_scatter(b)

        compute_chunk(b)
        pltpu.async_copy(msgs[b], acc_sh.at[dst_v.at[base + b]], ss[b],
                         add=True)

        @pl.when(base + b + NBUF_B < NCH)
        def _():
          issue_gather(base + b + NBUF_B, b)
      return carry
    lax.fori_loop(0, ngrp, grp_body, 0)
    for b in range(NBUF_B):
      wait_scatter(b)

    plsc.subcore_barrier()

    for kk in range(STRIPE // ZCH):
      base = sid * STRIPE + kk * ZCH
      pltpu.sync_copy(acc_sh.at[pl.ds(base, ZCH)], exp_v)
      pltpu.sync_copy(exp_v, out_hbm.at[cid].at[pl.ds(base, ZCH)])

  return k(tbl1, adt1, src2d, dst2d)


def _layer1_finish_body(acc_ref, tbl_ref, adt_ref, w2e_ref, b1_ref, r_ref,
                        tbl2_ref, adt2_ref):
  a = acc_ref[0] + acc_ref[1]
  as8 = tbl_ref[:, :8]
  h = tbl_ref[:, 8:]
  e = as8 + adt_ref[:, :8]
  ex8 = jnp.exp(jnp.maximum(e, 0.2 * e))
  den8 = a[:, :8] + ex8
  rmat = r_ref[...]
  exb = jnp.dot(ex8, rmat, preferred_element_type=jnp.float32)
  denb = jnp.dot(den8, rmat, preferred_element_type=jnp.float32)
  num = a[:, 8:] + h * exb
  out1 = num / (denb + 1e-16) + b1_ref[...]
  h2 = jnp.where(out1 > 0, out1, jnp.exp(out1) - 1.0)
  zc = jnp.dot(h2, w2e_ref[...], preferred_element_type=jnp.float32)
  z = zc[:, :NCLS]
  as2 = zc[:, NCLS:NCLS + 1]
  ad2 = zc[:, NCLS + 1:NCLS + 2]
  one = jnp.ones_like(as2)
  zero7 = jnp.zeros_like(z)
  tbl2_ref[...] = jnp.concatenate([one, z, as2, zero7], axis=1)
  adt2_ref[...] = ad2 * jnp.ones((1, 16), jnp.float32)


def _layer1_finish(acc1, tbl1, adt1, w2e, b1, rmat):
  blk = 2000
  return pl.pallas_call(
      _layer1_finish_body,
      grid=(N // blk,),
      in_specs=[
          pl.BlockSpec((NC, blk, TW), lambda i: (0, i, 0)),
          pl.BlockSpec((blk, TW), lambda i: (i, 0)),
          pl.BlockSpec((blk, 16), lambda i: (i, 0)),
          pl.BlockSpec((64, NCLS + 2), lambda i: (0, 0)),
          pl.BlockSpec((1, 64), lambda i: (0, 0)),
          pl.BlockSpec((8, 64), lambda i: (0, 0)),
      ],
      out_specs=[
          pl.BlockSpec((blk, 16), lambda i: (i, 0)),
          pl.BlockSpec((blk, 16), lambda i: (i, 0)),
      ],
      out_shape=[
          jax.ShapeDtypeStruct((N, 16), jnp.float32),
          jax.ShapeDtypeStruct((N, 16), jnp.float32),
      ],
  )(acc1, tbl1, adt1, w2e, b1, rmat)


def _edge_pass2(tbl2, adt2, src2d, dst2d):
  @functools.partial(
      pl.kernel,
      out_type=jax.ShapeDtypeStruct((NC, NACC, 16), jnp.float32),
      mesh=_mesh,
      compiler_params=_sc_params,
      scratch_types=[
          pltpu.VMEM((NCH, C), jnp.int32),
          pltpu.VMEM((NCH, C), jnp.int32),
          *([pltpu.VMEM((C, 16), jnp.float32)] * (3 * NBUF_D)),
          pltpu.VMEM((ZCH, 16), jnp.float32),
          pltpu.VMEM_SHARED((NACC, 16), jnp.float32),
          *([pltpu.SemaphoreType.DMA] * (3 * NBUF_D)),
      ],
  )
  def k(tbl_hbm, adt_hbm, src_hbm, dst_hbm, out_hbm,
        *refs):
    src_v, dst_v = refs[0], refs[1]
    rows = refs[2:2 + NBUF_D]
    ads = refs[2 + NBUF_D:2 + 2 * NBUF_D]
    msgs = refs[2 + 2 * NBUF_D:2 + 3 * NBUF_D]
    exp_v = refs[2 + 3 * NBUF_D]
    acc_sh = refs[3 + 3 * NBUF_D]
    sgr = refs[4 + 3 * NBUF_D:4 + 4 * NBUF_D]
    sga = refs[4 + 4 * NBUF_D:4 + 5 * NBUF_D]
    ss = refs[4 + 5 * NBUF_D:4 + 6 * NBUF_D]
    cid = lax.axis_index("c")
    sid = lax.axis_index("s")
    wid = sid * NC + cid
    lanes = lax.iota(jnp.int32, 16)
    idx8 = lanes * 0 + 8

    def zbody(i, carry):
      exp_v[i, :] = jnp.zeros((16,), jnp.float32)
      return carry
    lax.fori_loop(0, ZCH, zbody, 0)
    for kk in range(STRIPE // ZCH):
      pltpu.sync_copy(exp_v, acc_sh.at[pl.ds(sid * STRIPE + kk * ZCH, ZCH)])
    plsc.subcore_barrier()

    pltpu.sync_copy(src_hbm.at[wid], src_v)
    pltpu.sync_copy(dst_hbm.at[wid], dst_v)

    def issue_gather(ch, b):
      pltpu.async_copy(tbl_hbm.at[src_v.at[ch]], rows[b], sgr[b])
      pltpu.async_copy(adt_hbm.at[dst_v.at[ch]], ads[b], sga[b])

    def wait_gather(b):
      pltpu.make_async_copy(tbl_hbm.at[src_v.at[0]], rows[b], sgr[b]).wait()
      pltpu.make_async_copy(adt_hbm.at[dst_v.at[0]], ads[b], sga[b]).wait()

    def wait_scatter(b):
      pltpu.make_async_copy(msgs[b], acc_sh.at[dst_v.at[0]], ss[b]).wait()

    def compute_chunk(b):
      @plsc.parallel_loop(0, C, 1, unroll=5)
      def ebody(i):
        u = rows[b][i, :]
        as2 = plsc.load_gather(rows[b], [lanes * 0 + i, idx8])
        e = as2 + ads[b][i, :]
        ex = jnp.exp(jnp.maximum(e, 0.2 * e))
        msgs[b][i, :] = ex * u

    for b in range(NBUF_D):
      issue_gather(b, b)
    ngrp = NCH // NBUF_D

    def grp_body(g, carry):
      base = g * NBUF_D
      for b in range(NBUF_D):
        wait_gather(b)

        @pl.when(g > 0)
        def _():
          wait_scatter(b)

        compute_chunk(b)
        pltpu.async_copy(msgs[b], acc_sh.at[dst_v.at[base + b]], ss[b],
                         add=True)

        @pl.when(base + b + NBUF_D < NCH)
        def _():
          issue_gather(base + b + NBUF_D, b)
      return carry
    lax.fori_loop(0, ngrp, grp_body, 0)
    for b in range(NBUF_D):
      wait_scatter(b)

    plsc.subcore_barrier()

    for kk in range(STRIPE // ZCH):
      base = sid * STRIPE + kk * ZCH
      pltpu.sync_copy(acc_sh.at[pl.ds(base, ZCH)], exp_v)
      pltpu.sync_copy(exp_v, out_hbm.at[cid].at[pl.ds(base, ZCH)])

  return k(tbl2, adt2, src2d, dst2d)


def _final_body(acc_ref, tbl2_ref, adt2_ref, b2_ref, out_ref):
  a = acc_ref[0] + acc_ref[1]
  e = tbl2_ref[:, 8:9] + adt2_ref[:, 0:1]
  ex = jnp.exp(jnp.maximum(e, 0.2 * e))
  a = a + ex * tbl2_ref[:, :16]
  den = a[:, 0:1]
  num = a[:, 1:8]
  out2 = num / (den + 1e-16) + b2_ref[...]
  m = jnp.max(out2, axis=1, keepdims=True)
  lse = jnp.log(jnp.sum(jnp.exp(out2 - m), axis=1, keepdims=True)) + m
  out_ref[...] = out2 - lse


def _final(acc2, tbl2, adt2, b2):
  blk = 2000
  return pl.pallas_call(
      _final_body,
      grid=(N // blk,),
      in_specs=[
          pl.BlockSpec((NC, blk, 16), lambda i: (0, i, 0)),
          pl.BlockSpec((blk, 16), lambda i: (i, 0)),
          pl.BlockSpec((blk, 16), lambda i: (i, 0)),
          pl.BlockSpec((1, NCLS), lambda i: (0, 0)),
      ],
      out_specs=pl.BlockSpec((blk, NCLS), lambda i: (i, 0)),
      out_shape=jax.ShapeDtypeStruct((N, NCLS), jnp.float32),
  )(acc2, tbl2, adt2, b2)


def kernel(x, edge_index, W1, a1_src, a1_dst, b1, W2, a2_src, a2_dst, b2):
  eyeh = jnp.eye(HEADS, dtype=jnp.float32)
  a_src = (a1_src[:, :, None] * eyeh[:, None, :]).reshape(64, 8)
  a_dst = (a1_dst[:, :, None] * eyeh[:, None, :]).reshape(64, 8)
  zpad = jnp.zeros((F_IN, 8), jnp.float32)
  wcat = jnp.concatenate(
      [W1 @ a_src, W1, W1 @ a_dst, zpad], axis=1)
  rmat = jnp.kron(eyeh, jnp.ones((1, HID), jnp.float32))
  w2e = jnp.concatenate([W2, W2 @ a2_src.T, W2 @ a2_dst.T], axis=1)

  src2d = edge_index[0].astype(jnp.int32).reshape(NW, NCH, C)
  dst2d = edge_index[1].astype(jnp.int32).reshape(NW, NCH, C)
  src2d, dst2d = lax.optimization_barrier((src2d, dst2d))

  tbl1, adt1 = _build_tables1(x, wcat)
  acc1 = _edge_pass1(tbl1, adt1, src2d, dst2d)
  tbl2, adt2 = _layer1_finish(acc1, tbl1, adt1, w2e, b1.reshape(1, 64), rmat)
  acc2 = _edge_pass2(tbl2, adt2, src2d, dst2d)
  return _final(acc2, tbl2, adt2, b2.reshape(1, NCLS))
